# Initial kernel scaffold; baseline (speedup 1.0000x reference)
import jax
import jax.numpy as jnp
from jax.experimental import pallas as pl


def kernel(per_atom_energies, species, atomic_energy_table):
    raise NotImplementedError("write your pallas kernel here")


# speedup vs baseline: 362.2555x; 362.2555x over previous
import functools

import jax
import jax.numpy as jnp
from jax import lax
from jax.experimental import pallas as pl
from jax.experimental.pallas import tpu as pltpu
from jax.experimental.pallas import tpu_sc as plsc

_SCALE = 1.5
_SHIFT = 0.3

_NC = 2
_NS = 16
_NW = _NC * _NS
_LANES = 16
_TABLE_PAD = 128


def _make_sc_call(n: int, chunk: int):
  assert n % (_NW * chunk) == 0
  per_tile = n // _NW
  n_chunks = per_tile // chunk
  mesh = plsc.VectorSubcoreMesh(
      core_axis_name="c", subcore_axis_name="s",
      num_cores=_NC, num_subcores=_NS)

  @functools.partial(
      pl.kernel,
      out_type=jax.ShapeDtypeStruct((n,), jnp.float32),
      mesh=mesh,
      compiler_params=pltpu.CompilerParams(needs_layout_passes=False),
      scratch_types=[
          pltpu.VMEM((_TABLE_PAD,), jnp.float32),
          pltpu.VMEM((chunk,), jnp.int32),
          pltpu.VMEM((chunk,), jnp.float32),
          pltpu.VMEM((chunk,), jnp.float32),
      ],
  )
  def sc_call(e_hbm, s_hbm, t_hbm, out_hbm, table_v, idx_v, e_v, o_v):
    wid = lax.axis_index("s") * _NC + lax.axis_index("c")
    base = wid * per_tile
    pltpu.sync_copy(t_hbm, table_v)

    def chunk_body(ci, _):
      off = base + ci * chunk
      pltpu.sync_copy(s_hbm.at[pl.ds(off, chunk)], idx_v)
      pltpu.sync_copy(e_hbm.at[pl.ds(off, chunk)], e_v)

      @plsc.parallel_loop(0, chunk, step=_LANES)
      def _(i):
        idx = idx_v[pl.ds(i, _LANES)]
        e = e_v[pl.ds(i, _LANES)]
        g = plsc.load_gather(table_v, [idx])
        o_v[pl.ds(i, _LANES)] = g + (e * _SCALE + _SHIFT)

      pltpu.sync_copy(o_v, out_hbm.at[pl.ds(off, chunk)])
      return 0

    lax.fori_loop(0, n_chunks, chunk_body, 0)

  return sc_call


@jax.jit
def kernel(per_atom_energies, species, atomic_energy_table):
  n = per_atom_energies.shape[0]
  idx = species.astype(jnp.int32)
  table = jnp.pad(atomic_energy_table.reshape(-1),
                  (0, _TABLE_PAD - atomic_energy_table.shape[0]))
  call = _make_sc_call(n, chunk=8192)
  return call(per_atom_energies, idx, table)


# speedup vs baseline: 620.5475x; 1.7130x over previous
import functools

import jax
import jax.numpy as jnp
from jax import lax
from jax.experimental import pallas as pl
from jax.experimental.pallas import tpu as pltpu
from jax.experimental.pallas import tpu_sc as plsc

_SCALE = 1.5
_SHIFT = 0.3

_NC = 2
_NS = 16
_NW = _NC * _NS
_LANES = 16
_TABLE_PAD = 128


def _make_sc_call(n: int, chunk: int):
  assert n % (_NW * chunk) == 0
  per_tile = n // _NW
  n_chunks = per_tile // chunk
  mesh = plsc.VectorSubcoreMesh(
      core_axis_name="c", subcore_axis_name="s",
      num_cores=_NC, num_subcores=_NS)

  @functools.partial(
      pl.kernel,
      out_type=jax.ShapeDtypeStruct((n,), jnp.float32),
      mesh=mesh,
      compiler_params=pltpu.CompilerParams(needs_layout_passes=False),
      scratch_types=[
          pltpu.VMEM((_TABLE_PAD,), jnp.float32),
          pltpu.VMEM((chunk,), jnp.int32),
          pltpu.VMEM((chunk,), jnp.int32),
          pltpu.VMEM((chunk,), jnp.float32),
          pltpu.VMEM((chunk,), jnp.float32),
          pltpu.VMEM((chunk,), jnp.float32),
          pltpu.VMEM((chunk,), jnp.float32),
          pltpu.SemaphoreType.DMA,
          pltpu.SemaphoreType.DMA,
          pltpu.SemaphoreType.DMA,
          pltpu.SemaphoreType.DMA,
      ],
  )
  def sc_call(e_hbm, s_hbm, t_hbm, out_hbm, table_v,
              idx0, idx1, e0, e1, o0, o1, si0, si1, so0, so1):
    wid = lax.axis_index("s") * _NC + lax.axis_index("c")
    base = wid * per_tile
    pltpu.sync_copy(t_hbm, table_v)

    idx = (idx0, idx1)
    ev = (e0, e1)
    ov = (o0, o1)
    sin = (si0, si1)
    sout = (so0, so1)
    din = [None, None]
    dout = [None, None]

    def issue_in(ci):
      b = ci % 2
      off = base + ci * chunk
      din[b] = (
          pltpu.async_copy(s_hbm.at[pl.ds(off, chunk)], idx[b], sin[b]),
          pltpu.async_copy(e_hbm.at[pl.ds(off, chunk)], ev[b], sin[b]),
      )

    issue_in(0)
    if n_chunks > 1:
      issue_in(1)

    for ci in range(n_chunks):
      b = ci % 2
      d1, d2 = din[b]
      d1.wait()
      d2.wait()
      if dout[b] is not None:
        dout[b].wait()
        dout[b] = None

      @plsc.parallel_loop(0, chunk, step=_LANES, unroll=4)
      def _(i):
        ii = idx[b][pl.ds(i, _LANES)]
        e = ev[b][pl.ds(i, _LANES)]
        g = plsc.load_gather(table_v, [ii])
        ov[b][pl.ds(i, _LANES)] = g + (e * _SCALE + _SHIFT)

      off = base + ci * chunk
      dout[b] = pltpu.async_copy(ov[b], out_hbm.at[pl.ds(off, chunk)], sout[b])
      if ci + 2 < n_chunks:
        issue_in(ci + 2)

    for b in range(2):
      if dout[b] is not None:
        dout[b].wait()

  return sc_call


@jax.jit
def kernel(per_atom_energies, species, atomic_energy_table):
  n = per_atom_energies.shape[0]
  idx = species.astype(jnp.int32)
  table = jnp.pad(atomic_energy_table.reshape(-1),
                  (0, _TABLE_PAD - atomic_energy_table.shape[0]))
  call = _make_sc_call(n, chunk=8192)
  return call(per_atom_energies, idx, table)


# speedup vs baseline: 642.8859x; 1.0360x over previous
import functools

import jax
import jax.numpy as jnp
from jax import lax
from jax.experimental import pallas as pl
from jax.experimental.pallas import tpu as pltpu
from jax.experimental.pallas import tpu_sc as plsc

_SCALE = 1.5
_SHIFT = 0.3

_NC = 2
_NS = 16
_NW = _NC * _NS
_LANES = 16


def _make_sc_call(n: int, n_table: int, chunk: int):
  assert n % (_NW * 2 * chunk) == 0
  per_tile = n // _NW
  n_chunks = per_tile // chunk
  mesh = plsc.VectorSubcoreMesh(
      core_axis_name="c", subcore_axis_name="s",
      num_cores=_NC, num_subcores=_NS)

  @functools.partial(
      pl.kernel,
      out_type=jax.ShapeDtypeStruct((n,), jnp.float32),
      mesh=mesh,
      compiler_params=pltpu.CompilerParams(needs_layout_passes=False),
      scratch_types=[
          pltpu.VMEM((n_table,), jnp.float32),
          pltpu.VMEM((chunk,), jnp.int32),
          pltpu.VMEM((chunk,), jnp.int32),
          pltpu.VMEM((chunk,), jnp.float32),
          pltpu.VMEM((chunk,), jnp.float32),
          pltpu.VMEM((chunk,), jnp.float32),
          pltpu.VMEM((chunk,), jnp.float32),
          pltpu.SemaphoreType.DMA,
          pltpu.SemaphoreType.DMA,
          pltpu.SemaphoreType.DMA,
          pltpu.SemaphoreType.DMA,
      ],
  )
  def sc_call(e_hbm, s_hbm, t_hbm, out_hbm, table_v,
              idx0, idx1, e0, e1, o0, o1, si0, si1, so0, so1):
    wid = lax.axis_index("s") * _NC + lax.axis_index("c")
    base = wid * per_tile
    pltpu.sync_copy(t_hbm, table_v)

    idx = (idx0, idx1)
    ev = (e0, e1)
    ov = (o0, o1)
    sin = (si0, si1)
    sout = (so0, so1)

    def issue_in(ci, b):
      off = base + ci * chunk
      pltpu.async_copy(s_hbm.at[pl.ds(off, chunk)], idx[b], sin[b])
      pltpu.async_copy(e_hbm.at[pl.ds(off, chunk)], ev[b], sin[b])

    def wait_in(b):
      pltpu.make_async_copy(s_hbm.at[pl.ds(0, chunk)], idx[b], sin[b]).wait()
      pltpu.make_async_copy(e_hbm.at[pl.ds(0, chunk)], ev[b], sin[b]).wait()

    def wait_out(b):
      pltpu.make_async_copy(ov[b], out_hbm.at[pl.ds(0, chunk)], sout[b]).wait()

    issue_in(0, 0)
    issue_in(1, 1)

    def slot_body(ci, b):
      wait_in(b)

      @pl.when(ci >= 2)
      def _():
        wait_out(b)

      @plsc.parallel_loop(0, chunk, step=_LANES, unroll=4)
      def _(i):
        ii = idx[b][pl.ds(i, _LANES)]
        e = ev[b][pl.ds(i, _LANES)]
        g = plsc.load_gather(table_v, [ii])
        ov[b][pl.ds(i, _LANES)] = g + (e * _SCALE + _SHIFT)

      pltpu.async_copy(ov[b], out_hbm.at[pl.ds(base + ci * chunk, chunk)],
                       sout[b])

      @pl.when(ci + 2 < n_chunks)
      def _():
        issue_in(ci + 2, b)

    def pair_body(k, carry):
      ci = k * 2
      slot_body(ci, 0)
      slot_body(ci + 1, 1)
      return carry

    lax.fori_loop(0, n_chunks // 2, pair_body, 0)
    wait_out(0)
    wait_out(1)

  return sc_call


@jax.jit
def kernel(per_atom_energies, species, atomic_energy_table):
  n = per_atom_energies.shape[0]
  idx = species.astype(jnp.int32)
  table = atomic_energy_table.reshape(-1)
  call = _make_sc_call(n, table.shape[0], chunk=8192)
  return call(per_atom_energies, idx, table)
